# baseline (speedup 1.0000x reference)
import functools

import jax
import jax.numpy as jnp
from jax import lax
from jax.experimental import pallas as pl
from jax.experimental.pallas import tpu as pltpu
from jax.experimental.pallas import tpu_sc as plsc

_VOCAB = 32002
_HIDDEN = 2048
_BATCH = 4
_SEQ = 4096
_NTOK = _BATCH * _SEQ
_NW = 32
_PER_W = _NTOK // _NW
_CHUNK = 24
_NSLOT = 22
_TAIL = _PER_W - 21 * _CHUNK

_mesh = plsc.VectorSubcoreMesh(core_axis_name="c", subcore_axis_name="s")


@functools.partial(
    pl.kernel,
    out_type=jax.ShapeDtypeStruct((_NTOK, _HIDDEN), jnp.float32),
    mesh=_mesh,
    scratch_types=[
        pltpu.VMEM((_NSLOT, _CHUNK), jnp.int32),
        pltpu.VMEM((_CHUNK, _HIDDEN), jnp.float32),
        pltpu.VMEM((_CHUNK, _HIDDEN), jnp.float32),
        pltpu.SemaphoreType.DMA,
        pltpu.SemaphoreType.DMA,
        pltpu.SemaphoreType.DMA,
        pltpu.SemaphoreType.DMA,
    ],
)
def _embed_lookup(table_hbm, idx_hbm, out_hbm, idx_v, buf0, buf1,
                  g0, g1, o0, o1):
    wid = lax.axis_index("s") * 2 + lax.axis_index("c")
    base = wid * _PER_W
    pltpu.sync_copy(idx_hbm.at[wid], idx_v)

    bufs = (buf0, buf1)
    gsems = (g0, g1)
    osems = (o0, o1)

    def gather_start(j, b):
        pltpu.async_copy(table_hbm.at[idx_v.at[j]], bufs[b], gsems[b])

    def gather_wait(b):
        pltpu.make_async_copy(
            table_hbm.at[idx_v.at[0]], bufs[b], gsems[b]).wait()

    def out_start(j, b):
        pltpu.async_copy(
            bufs[b], out_hbm.at[pl.ds(base + j * _CHUNK, _CHUNK)], osems[b])

    def out_wait(b):
        pltpu.make_async_copy(
            bufs[b], out_hbm.at[pl.ds(base, _CHUNK)], osems[b]).wait()

    gather_start(0, 0)
    gather_start(1, 1)
    gather_wait(0)
    out_start(0, 0)
    gather_wait(1)
    out_start(1, 1)

    def step(k, carry):
        for b in range(2):
            j = 2 * k + b
            out_wait(b)
            gather_start(j, b)
            gather_wait(b)
            out_start(j, b)
        return carry

    lax.fori_loop(1, (_NSLOT - 1) // 2, step, 0)

    out_wait(0)
    gather_start(_NSLOT - 2, 0)
    gather_wait(0)
    out_start(_NSLOT - 2, 0)

    out_wait(1)
    gather_start(_NSLOT - 1, 1)
    gather_wait(1)
    pltpu.async_copy(
        buf1.at[pl.ds(0, _TAIL)],
        out_hbm.at[pl.ds(base + (_NSLOT - 1) * _CHUNK, _TAIL)], o1)
    out_wait(0)
    pltpu.make_async_copy(
        buf1.at[pl.ds(0, _TAIL)],
        out_hbm.at[pl.ds(base, _TAIL)], o1).wait()


def kernel(input_ids, embedding_table):
    ids = input_ids.reshape(_NW, _PER_W)
    pad = _NSLOT * _CHUNK - _PER_W
    ids = jnp.concatenate([ids, ids[:, -pad:]], axis=1)
    ids = ids.reshape(_NW, _NSLOT, _CHUNK)
    out = _embed_lookup(embedding_table, ids)
    return out.reshape(_BATCH, _SEQ, _HIDDEN)
